# baseline (speedup 1.0000x reference)
import functools

import jax
import jax.numpy as jnp
from jax import lax
from jax.experimental import pallas as pl
from jax.experimental.pallas import tpu as pltpu
from jax.experimental.pallas import tpu_sc as plsc

N_USERS = 1000000
N_ITEMS = 1000000
D = 64
B = 16384

NC = 2
NS = 16
NW = NC * NS
BPW = B // NW
CHUNK = 32
NCH = BPW // CHUNK


@functools.partial(
    pl.kernel,
    out_type=jax.ShapeDtypeStruct((B,), jnp.float32),
    mesh=plsc.VectorSubcoreMesh(core_axis_name="c", subcore_axis_name="s"),
    compiler_params=pltpu.CompilerParams(
        needs_layout_passes=False, use_tc_tiling_on_sc=True),
    scratch_types=[
        pltpu.VMEM((BPW,), jnp.int32),
        pltpu.VMEM((BPW,), jnp.int32),
        pltpu.VMEM((CHUNK, 128), jnp.float32),
        pltpu.VMEM((CHUNK, 128), jnp.float32),
        pltpu.VMEM((BPW,), jnp.float32),
        pltpu.SemaphoreType.DMA,
    ],
)
def _lfm_sc(users_h, items_h, uep, iep, out,
            uidx_v, iidx_v, ug_v, ig_v, out_v, sem):
    wid = lax.axis_index("s") * NC + lax.axis_index("c")
    base = wid * BPW

    pltpu.sync_copy(users_h.at[pl.ds(base, BPW)], uidx_v)
    pltpu.sync_copy(items_h.at[pl.ds(base, BPW)], iidx_v)

    lane = lax.iota(jnp.int32, 16)
    mask2 = (lane < 2).astype(jnp.float32)

    def chunk_body(c, carry):
        cbase = c * CHUNK
        csl = pl.ds(cbase, CHUNK)
        cps = [
            pltpu.async_copy(uep.at[uidx_v.at[csl]], ug_v, sem),
            pltpu.async_copy(iep.at[iidx_v.at[csl]], ig_v, sem),
        ]
        for cp in cps:
            cp.wait()

        for g in range(CHUNK // 16):
            sl = pl.ds(cbase + g * 16, 16)
            tot = jnp.zeros((16,), jnp.float32)
            for l in range(16):
                r = g * 16 + l
                acc = (ug_v[r, pl.ds(0, 16)] * ig_v[r, pl.ds(0, 16)]
                       + ug_v[r, pl.ds(16, 16)] * ig_v[r, pl.ds(16, 16)])
                acc = acc + (ug_v[r, pl.ds(32, 16)] * ig_v[r, pl.ds(32, 16)]
                             + ug_v[r, pl.ds(48, 16)] * ig_v[r, pl.ds(48, 16)])
                acc = acc + (ug_v[r, pl.ds(64, 16)] * ig_v[r, pl.ds(64, 16)]
                             * mask2)
                tot = jnp.where(lane == l, jnp.sum(acc), tot)
            out_v[sl] = tot
        return carry

    lax.fori_loop(0, NCH, chunk_body, 0)

    pltpu.sync_copy(out_v, out.at[pl.ds(base, BPW)])


def kernel(users, items, user_embeddings, item_embeddings, user_biases, item_biases):
    ones = jnp.ones((N_USERS, 1), jnp.float32)
    zeros = jnp.zeros((N_USERS, 128 - D - 2), jnp.float32)
    uep = jnp.concatenate([user_embeddings, user_biases, ones, zeros], axis=1)
    iep = jnp.concatenate([item_embeddings, ones, item_biases, zeros], axis=1)
    return _lfm_sc(users.astype(jnp.int32), items.astype(jnp.int32), uep, iep)


# speedup vs baseline: 2.0263x; 2.0263x over previous
import functools

import jax
import jax.numpy as jnp
from jax import lax
from jax.experimental import pallas as pl
from jax.experimental.pallas import tpu as pltpu
from jax.experimental.pallas import tpu_sc as plsc

N_USERS = 1000000
N_ITEMS = 1000000
D = 64
B = 16384

NC = 2
NS = 16
NW = NC * NS
BPW = B // NW
CHUNK = 32
NCH = BPW // CHUNK
NBROW = (N_USERS + 127) // 128


@functools.partial(
    pl.kernel,
    out_type=jax.ShapeDtypeStruct((B,), jnp.float32),
    mesh=plsc.VectorSubcoreMesh(core_axis_name="c", subcore_axis_name="s"),
    compiler_params=pltpu.CompilerParams(
        needs_layout_passes=False, use_tc_tiling_on_sc=True),
    scratch_types=[
        pltpu.VMEM((BPW,), jnp.int32),
        pltpu.VMEM((BPW,), jnp.int32),
        pltpu.VMEM((BPW,), jnp.int32),
        pltpu.VMEM((BPW,), jnp.int32),
        pltpu.VMEM((CHUNK, 128), jnp.float32),
        pltpu.VMEM((CHUNK, 128), jnp.float32),
        pltpu.VMEM((CHUNK, 128), jnp.float32),
        pltpu.VMEM((CHUNK, 128), jnp.float32),
        pltpu.VMEM((BPW,), jnp.float32),
        pltpu.SemaphoreType.DMA,
    ],
)
def _lfm_sc(users_h, items_h, uep, iep, ubp, ibp, out,
            uidx_v, iidx_v, ubr_v, ibr_v,
            ug_v, ig_v, ubg_v, ibg_v, out_v, sem):
    wid = lax.axis_index("s") * NC + lax.axis_index("c")
    base = wid * BPW

    pltpu.sync_copy(users_h.at[pl.ds(base, BPW)], uidx_v)
    pltpu.sync_copy(items_h.at[pl.ds(base, BPW)], iidx_v)

    def derive(j, carry):
        sl = pl.ds(j * 16, 16)
        ubr_v[sl] = lax.shift_right_logical(uidx_v[sl], 7)
        ibr_v[sl] = lax.shift_right_logical(iidx_v[sl], 7)
        return carry

    lax.fori_loop(0, BPW // 16, derive, 0)

    lane = lax.iota(jnp.int32, 16)

    def chunk_body(c, carry):
        cbase = c * CHUNK
        csl = pl.ds(cbase, CHUNK)
        cps = [
            pltpu.async_copy(uep.at[uidx_v.at[csl]], ug_v, sem),
            pltpu.async_copy(iep.at[iidx_v.at[csl]], ig_v, sem),
            pltpu.async_copy(ubp.at[ubr_v.at[csl]], ubg_v, sem),
            pltpu.async_copy(ibp.at[ibr_v.at[csl]], ibg_v, sem),
        ]
        for cp in cps:
            cp.wait()

        for g in range(CHUNK // 16):
            sl = pl.ds(cbase + g * 16, 16)
            uvec = uidx_v[sl]
            ivec = iidx_v[sl]
            row16 = g * 16 + lane
            tot = plsc.load_gather(ubg_v, [row16, uvec & 127])
            tot = tot + plsc.load_gather(ibg_v, [row16, ivec & 127])
            for l in range(16):
                r = g * 16 + l
                acc = (ug_v[r, pl.ds(0, 16)] * ig_v[r, pl.ds(0, 16)]
                       + ug_v[r, pl.ds(16, 16)] * ig_v[r, pl.ds(16, 16)])
                acc = acc + (ug_v[r, pl.ds(32, 16)] * ig_v[r, pl.ds(32, 16)]
                             + ug_v[r, pl.ds(48, 16)] * ig_v[r, pl.ds(48, 16)])
                tot = jnp.where(lane == l, tot + jnp.sum(acc), tot)
            out_v[sl] = tot
        return carry

    lax.fori_loop(0, NCH, chunk_body, 0)

    pltpu.sync_copy(out_v, out.at[pl.ds(base, BPW)])


def kernel(users, items, user_embeddings, item_embeddings, user_biases, item_biases):
    uep = jnp.pad(user_embeddings, ((0, 0), (0, 128 - D)))
    iep = jnp.pad(item_embeddings, ((0, 0), (0, 128 - D)))
    ubp = jnp.pad(user_biases.reshape(N_USERS),
                  (0, NBROW * 128 - N_USERS)).reshape(NBROW, 128)
    ibp = jnp.pad(item_biases.reshape(N_ITEMS),
                  (0, NBROW * 128 - N_ITEMS)).reshape(NBROW, 128)
    return _lfm_sc(users.astype(jnp.int32), items.astype(jnp.int32),
                   uep, iep, ubp, ibp)
